# baseline (speedup 1.0000x reference)
import functools

import numpy as np
import jax
import jax.numpy as jnp
from jax import lax
from jax.experimental import pallas as pl
from jax.experimental.pallas import tpu as pltpu
from jax.experimental.pallas import tpu_sc as plsc

K = 60
KP = 64
BIG = np.float32(1e10)

SC_CORES = 2
SC_SUBCORES = 16
SC_WORKERS = SC_CORES * SC_SUBCORES


def _knn_qp_body(posr_ref, posrt_ref, posc_ref, batrt_ref, batc_ref,
                 batt_ref, x_ref, w1a_ref, w1b_ref, b1_ref,
                 idx_ref, q_ref, p_ref, d_ref, *, n, r, c):
    i = pl.program_id(0)

    p = jnp.dot(posr_ref[...], w1b_ref[...],
                preferred_element_type=jnp.float32)
    q = jnp.dot(x_ref[...], w1a_ref[...],
                preferred_element_type=jnp.float32) + p + b1_ref[...]
    p_ref[...] = p
    q_ref[...] = q

    batrt = batrt_ref[...]
    rowlane = i * r + lax.broadcasted_iota(jnp.int32, (1, r), 1)

    inf = jnp.full((1, r), jnp.inf, jnp.float32)
    nfull = jnp.full((1, r), n, jnp.int32)

    coln = lax.broadcasted_iota(jnp.int32, (1, n), 1)
    batt = batt_ref[...]
    b_first = batrt_ref[0, 0]
    b_last = batrt_ref[0, r - 1]
    lo = jnp.min(jnp.where(batt == b_first, coln, n))
    hi = jnp.max(jnp.where(batt == b_last, coln, -1)) + 1
    lo_al0 = (lo // c) * c
    nch_w = (hi - lo_al0 + (c - 1)) // c
    nmax = jnp.int32(n - c)

    def make_build_step(lo_base):
        def build_step(ci, carry):
            m0, j0, cnt = carry
            start = jnp.minimum(lo_base + ci * c, nmax)
            sl = pl.ds(ci * c, c)
            src = pl.ds(start, c)
            colc = start + lax.broadcasted_iota(jnp.int32, (c, r), 0)
            dx = posc_ref[src, 0:1] - posrt_ref[0:1, :]
            dy = posc_ref[src, 1:2] - posrt_ref[1:2, :]
            dz = posc_ref[src, 2:3] - posrt_ref[2:3, :]
            d = (dx * dx + dy * dy) + dz * dz
            same = batc_ref[src, 0:1] == batrt
            d = jnp.where((~same) | (colc == rowlane), BIG, d)
            d_ref[sl, :] = d
            cm = jnp.min(d, axis=0, keepdims=True)
            cj = jnp.min(jnp.where(d == cm, colc, n), axis=0, keepdims=True)
            upd = cm < m0
            m0 = jnp.where(upd, cm, m0)
            j0 = jnp.where(upd, cj, j0)
            cnt = cnt + jnp.sum(same.astype(jnp.int32), axis=0,
                                keepdims=True)
            return m0, j0, cnt

        return build_step

    zero_cnt = jnp.zeros((1, r), jnp.int32)
    m0, j0, cnt = lax.fori_loop(0, nch_w, make_build_step(lo_al0),
                                (inf, nfull, zero_cnt))

    degen = jnp.min(cnt) < (K + 1)
    lo_al = jnp.where(degen, 0, lo_al0)
    nch_full = jnp.where(degen, jnp.int32(n // c), 0)
    m0f, j0f, _ = lax.fori_loop(0, nch_full, make_build_step(lo_al),
                                (inf, nfull, zero_cnt))
    m0 = jnp.where(degen, m0f, m0)
    j0 = jnp.where(degen, j0f, j0)
    nch = jnp.where(degen, jnp.int32(n // c), nch_w)

    subl = lax.broadcasted_iota(jnp.int32, (KP, r), 0)
    acc0 = jnp.where(subl == 0, j0,
                     jnp.where(subl >= K, jnp.broadcast_to(rowlane, (KP, r)),
                               0))

    def select_step(t, carry):
        jprev, acc = carry

        def scan_chunk(ci, inner):
            m, j = inner
            start = jnp.minimum(lo_al + ci * c, nmax)
            sl = pl.ds(ci * c, c)
            colc = start + lax.broadcasted_iota(jnp.int32, (c, r), 0)
            d = d_ref[sl, :]
            d = jnp.where(colc == jprev, np.float32(np.inf), d)
            d_ref[sl, :] = d
            cm = jnp.min(d, axis=0, keepdims=True)
            cj = jnp.min(jnp.where(d == cm, colc, n), axis=0, keepdims=True)
            upd = cm < m
            m = jnp.where(upd, cm, m)
            j = jnp.where(upd, cj, j)
            return m, j

        m, j = lax.fori_loop(0, nch, scan_chunk, (inf, nfull))
        acc = jnp.where(subl == t, j, acc)
        return j, acc

    _, accK = lax.fori_loop(1, K, select_step, (j0, acc0))
    idx_ref[...] = accK


def _run_knn_qp(pos, batch_i32, x, w1a, w1b, b1, *, n, h, r, c):
    post = pos.T
    batc = batch_i32.reshape(n, 1)
    batt = batch_i32.reshape(1, n)
    grid = (n // r,)
    kern = pl.pallas_call(
        functools.partial(_knn_qp_body, n=n, r=r, c=c),
        grid=grid,
        in_specs=[
            pl.BlockSpec((r, 3), lambda i: (i, 0)),
            pl.BlockSpec((3, r), lambda i: (0, i)),
            pl.BlockSpec((n, 3), lambda i: (0, 0)),
            pl.BlockSpec((1, r), lambda i: (0, i)),
            pl.BlockSpec((n, 1), lambda i: (0, 0)),
            pl.BlockSpec((1, n), lambda i: (0, 0)),
            pl.BlockSpec((r, x.shape[1]), lambda i: (i, 0)),
            pl.BlockSpec(w1a.shape, lambda i: (0, 0)),
            pl.BlockSpec(w1b.shape, lambda i: (0, 0)),
            pl.BlockSpec((1, h), lambda i: (0, 0)),
        ],
        out_specs=[
            pl.BlockSpec((KP, r), lambda i: (0, i)),
            pl.BlockSpec((r, h), lambda i: (i, 0)),
            pl.BlockSpec((r, h), lambda i: (i, 0)),
        ],
        out_shape=[
            jax.ShapeDtypeStruct((KP, n), jnp.int32),
            jax.ShapeDtypeStruct((n, h), jnp.float32),
            jax.ShapeDtypeStruct((n, h), jnp.float32),
        ],
        scratch_shapes=[pltpu.VMEM((n, r), jnp.float32)],
        compiler_params=pltpu.CompilerParams(
            dimension_semantics=("arbitrary",)),
    )
    return kern(pos, post, pos, batt, batc, batt, x, w1a, w1b,
                b1.reshape(1, h))


def _sc_gather(q, src_flat, *, n, h):
    b = src_flat.shape[0]
    b_per_w = b // SC_WORKERS
    ch = 512
    nch = b_per_w // ch
    mesh = plsc.VectorSubcoreMesh(core_axis_name="c", subcore_axis_name="s")

    @functools.partial(
        pl.kernel,
        mesh=mesh,
        out_type=jax.ShapeDtypeStruct((b, h), jnp.float32),
        scratch_types=[
            pltpu.VMEM((ch,), jnp.int32),
            pltpu.VMEM((ch, h), jnp.float32),
            pltpu.SemaphoreType.DMA,
        ],
    )
    def gather_kernel(table_hbm, idx_hbm, out_hbm, idx_v, rows_v, sem):
        wid = lax.axis_index("s") * SC_CORES + lax.axis_index("c")
        base = wid * b_per_w

        def body(ci, carry):
            off = base + ci * ch
            pltpu.sync_copy(idx_hbm.at[pl.ds(off, ch)], idx_v)
            pltpu.async_copy(table_hbm.at[idx_v], rows_v, sem).wait()
            pltpu.sync_copy(rows_v, out_hbm.at[pl.ds(off, ch)])
            return carry

        lax.fori_loop(0, nch, body, 0)

    return gather_kernel(q, src_flat)


def _mlp_max_body(qe_ref, p_ref, w2_ref, b2_ref, out_ref, *, t, h):
    pre = qe_ref[...] - p_ref[...][None, :, :]
    pre = jnp.maximum(pre, jnp.float32(0.0))
    hm = jnp.dot(pre.reshape(KP * t, h), w2_ref[...],
                 preferred_element_type=jnp.float32)
    hm = jnp.max(hm.reshape(KP, t, h), axis=0)
    out_ref[...] = hm + b2_ref[...]


def _run_mlp_max(qe3, p, w2, b2, *, n, h, t):
    grid = (n // t,)
    kern = pl.pallas_call(
        functools.partial(_mlp_max_body, t=t, h=h),
        grid=grid,
        in_specs=[
            pl.BlockSpec((KP, t, h), lambda i: (0, i, 0)),
            pl.BlockSpec((t, h), lambda i: (i, 0)),
            pl.BlockSpec((h, h), lambda i: (0, 0)),
            pl.BlockSpec((1, h), lambda i: (0, 0)),
        ],
        out_specs=pl.BlockSpec((t, h), lambda i: (i, 0)),
        out_shape=jax.ShapeDtypeStruct((n, h), jnp.float32),
        compiler_params=pltpu.CompilerParams(
            dimension_semantics=("arbitrary",)),
    )
    return kern(qe3, p, w2, b2.reshape(1, h))


def kernel(x, pos, batch, W1, b1, W2, b2):
    n, d = x.shape
    h = W1.shape[1]
    batch_i32 = batch.astype(jnp.int32)
    w1a = W1[:d]
    w1b = W1[d:]

    idx_t, q, p = _run_knn_qp(pos, batch_i32, x, w1a, w1b, b1,
                              n=n, h=h, r=min(256, n), c=min(128, n))

    src_flat = idx_t.reshape(-1)
    qe = _sc_gather(q, src_flat, n=n, h=h)
    out = _run_mlp_max(qe.reshape(KP, n, h), p, W2, b2, n=n, h=h,
                       t=min(128, n))

    src = idx_t[:K].T.reshape(-1)
    dst = jnp.broadcast_to(jnp.arange(n, dtype=jnp.int32)[:, None],
                           (n, K)).reshape(-1)
    edge_index = jnp.stack([src, dst])
    return (out, pos, batch, edge_index)


# speedup vs baseline: 1.0906x; 1.0906x over previous
import functools

import numpy as np
import jax
import jax.numpy as jnp
from jax import lax
from jax.experimental import pallas as pl
from jax.experimental.pallas import tpu as pltpu
from jax.experimental.pallas import tpu_sc as plsc

K = 60
KP = 64
BIG = np.float32(1e10)

SC_CORES = 2
SC_SUBCORES = 16
SC_WORKERS = SC_CORES * SC_SUBCORES


def _knn_qp_body(posr_ref, posrt_ref, posc_ref, batrt_ref, batc_ref,
                 batt_ref, x_ref, w1a_ref, w1b_ref, b1_ref,
                 idx_ref, q_ref, p_ref, d_ref, *, n, r, c):
    i = pl.program_id(0)

    p = jnp.dot(posr_ref[...], w1b_ref[...],
                preferred_element_type=jnp.float32)
    q = jnp.dot(x_ref[...], w1a_ref[...],
                preferred_element_type=jnp.float32) + p + b1_ref[...]
    p_ref[...] = p
    q_ref[...] = q

    batrt = batrt_ref[...]
    rowlane = i * r + lax.broadcasted_iota(jnp.int32, (1, r), 1)

    inf = jnp.full((1, r), jnp.inf, jnp.float32)
    nfull = jnp.full((1, r), n, jnp.int32)

    coln = lax.broadcasted_iota(jnp.int32, (1, n), 1)
    batt = batt_ref[...]
    b_first = batrt_ref[0, 0]
    b_last = batrt_ref[0, r - 1]
    lo = jnp.min(jnp.where(batt == b_first, coln, n))
    hi = jnp.max(jnp.where(batt == b_last, coln, -1)) + 1
    lo_al0 = (lo // c) * c
    nch_w = (hi - lo_al0 + (c - 1)) // c
    nmax = jnp.int32(n - c)

    def make_build_step(lo_base):
        def build_step(ci, carry):
            m0, j0, cnt = carry
            start = jnp.minimum(lo_base + ci * c, nmax)
            sl = pl.ds(ci * c, c)
            src = pl.ds(start, c)
            colc = start + lax.broadcasted_iota(jnp.int32, (c, r), 0)
            dx = posc_ref[src, 0:1] - posrt_ref[0:1, :]
            dy = posc_ref[src, 1:2] - posrt_ref[1:2, :]
            dz = posc_ref[src, 2:3] - posrt_ref[2:3, :]
            d = (dx * dx + dy * dy) + dz * dz
            same = batc_ref[src, 0:1] == batrt
            d = jnp.where((~same) | (colc == rowlane), BIG, d)
            d_ref[sl, :] = d
            cm = jnp.min(d, axis=0, keepdims=True)
            cj = jnp.min(jnp.where(d == cm, colc, n), axis=0, keepdims=True)
            upd = cm < m0
            m0 = jnp.where(upd, cm, m0)
            j0 = jnp.where(upd, cj, j0)
            cnt = cnt + jnp.sum(same.astype(jnp.int32), axis=0,
                                keepdims=True)
            return m0, j0, cnt

        return build_step

    zero_cnt = jnp.zeros((1, r), jnp.int32)
    m0, j0, cnt = lax.fori_loop(0, nch_w, make_build_step(lo_al0),
                                (inf, nfull, zero_cnt))

    degen = jnp.min(cnt) < (K + 1)
    lo_al = jnp.where(degen, 0, lo_al0)
    nch_full = jnp.where(degen, jnp.int32(n // c), 0)
    m0f, j0f, _ = lax.fori_loop(0, nch_full, make_build_step(lo_al),
                                (inf, nfull, zero_cnt))
    m0 = jnp.where(degen, m0f, m0)
    j0 = jnp.where(degen, j0f, j0)
    nch = jnp.where(degen, jnp.int32(n // c), nch_w)

    subl = lax.broadcasted_iota(jnp.int32, (KP, r), 0)
    acc0 = jnp.where(subl == 0, j0,
                     jnp.where(subl >= K, jnp.broadcast_to(rowlane, (KP, r)),
                               0))

    def select_step(t, carry):
        jprev, acc = carry

        def scan_chunk(ci, inner):
            m, j = inner
            start = jnp.minimum(lo_al + ci * c, nmax)
            sl = pl.ds(ci * c, c)
            colc = start + lax.broadcasted_iota(jnp.int32, (c, r), 0)
            d = d_ref[sl, :]
            d = jnp.where(colc == jprev, np.float32(np.inf), d)
            d_ref[sl, :] = d
            cm = jnp.min(d, axis=0, keepdims=True)
            cj = jnp.min(jnp.where(d == cm, colc, n), axis=0, keepdims=True)
            upd = cm < m
            m = jnp.where(upd, cm, m)
            j = jnp.where(upd, cj, j)
            return m, j

        m, j = lax.fori_loop(0, nch, scan_chunk, (inf, nfull))
        acc = jnp.where(subl == t, j, acc)
        return j, acc

    _, accK = lax.fori_loop(1, K, select_step, (j0, acc0))
    idx_ref[...] = accK


def _run_knn_qp(pos, batch_i32, x, w1a, w1b, b1, *, n, h, r, c):
    post = pos.T
    batc = batch_i32.reshape(n, 1)
    batt = batch_i32.reshape(1, n)
    grid = (n // r,)
    kern = pl.pallas_call(
        functools.partial(_knn_qp_body, n=n, r=r, c=c),
        grid=grid,
        in_specs=[
            pl.BlockSpec((r, 3), lambda i: (i, 0)),
            pl.BlockSpec((3, r), lambda i: (0, i)),
            pl.BlockSpec((n, 3), lambda i: (0, 0)),
            pl.BlockSpec((1, r), lambda i: (0, i)),
            pl.BlockSpec((n, 1), lambda i: (0, 0)),
            pl.BlockSpec((1, n), lambda i: (0, 0)),
            pl.BlockSpec((r, x.shape[1]), lambda i: (i, 0)),
            pl.BlockSpec(w1a.shape, lambda i: (0, 0)),
            pl.BlockSpec(w1b.shape, lambda i: (0, 0)),
            pl.BlockSpec((1, h), lambda i: (0, 0)),
        ],
        out_specs=[
            pl.BlockSpec((KP, r), lambda i: (0, i)),
            pl.BlockSpec((r, h), lambda i: (i, 0)),
            pl.BlockSpec((r, h), lambda i: (i, 0)),
        ],
        out_shape=[
            jax.ShapeDtypeStruct((KP, n), jnp.int32),
            jax.ShapeDtypeStruct((n, h), jnp.float32),
            jax.ShapeDtypeStruct((n, h), jnp.float32),
        ],
        scratch_shapes=[pltpu.VMEM((n, r), jnp.float32)],
        compiler_params=pltpu.CompilerParams(
            dimension_semantics=("arbitrary",)),
    )
    return kern(pos, post, pos, batt, batc, batt, x, w1a, w1b,
                b1.reshape(1, h))


def _sc_gather(q, src_flat, *, n, h):
    b = src_flat.shape[0]
    b_per_w = b // SC_WORKERS
    ch = 512
    nch = b_per_w // ch
    mesh = plsc.VectorSubcoreMesh(core_axis_name="c", subcore_axis_name="s")

    @functools.partial(
        pl.kernel,
        mesh=mesh,
        out_type=jax.ShapeDtypeStruct((b, h), jnp.float32),
        scratch_types=[
            pltpu.VMEM((ch,), jnp.int32),
            pltpu.VMEM((ch, h), jnp.float32),
            pltpu.SemaphoreType.DMA,
        ],
    )
    def gather_kernel(table_hbm, idx_hbm, out_hbm, idx_v, rows_v, sem):
        wid = lax.axis_index("s") * SC_CORES + lax.axis_index("c")
        base = wid * b_per_w

        def body(ci, carry):
            off = base + ci * ch
            pltpu.sync_copy(idx_hbm.at[pl.ds(off, ch)], idx_v)
            pltpu.async_copy(table_hbm.at[idx_v], rows_v, sem).wait()
            pltpu.sync_copy(rows_v, out_hbm.at[pl.ds(off, ch)])
            return carry

        lax.fori_loop(0, nch, body, 0)

    return gather_kernel(q, src_flat)


def _mlp_max_body(qe_ref, p_ref, w2_ref, b2_ref, out_ref, *, t, h):
    pre = qe_ref[...] - p_ref[...][None, :, :]
    pre = jnp.maximum(pre, jnp.float32(0.0))
    hm = jnp.dot(pre.reshape(KP * t, h), w2_ref[...],
                 preferred_element_type=jnp.float32)
    hm = jnp.max(hm.reshape(KP, t, h), axis=0)
    out_ref[...] = hm + b2_ref[...]


def _run_mlp_max(qe3, p, w2, b2, *, n, h, t):
    grid = (n // t,)
    kern = pl.pallas_call(
        functools.partial(_mlp_max_body, t=t, h=h),
        grid=grid,
        in_specs=[
            pl.BlockSpec((KP, t, h), lambda i: (0, i, 0)),
            pl.BlockSpec((t, h), lambda i: (i, 0)),
            pl.BlockSpec((h, h), lambda i: (0, 0)),
            pl.BlockSpec((1, h), lambda i: (0, 0)),
        ],
        out_specs=pl.BlockSpec((t, h), lambda i: (i, 0)),
        out_shape=jax.ShapeDtypeStruct((n, h), jnp.float32),
        compiler_params=pltpu.CompilerParams(
            dimension_semantics=("arbitrary",)),
    )
    return kern(qe3, p, w2, b2.reshape(1, h))


def kernel(x, pos, batch, W1, b1, W2, b2):
    n, d = x.shape
    h = W1.shape[1]
    batch_i32 = batch.astype(jnp.int32)
    w1a = W1[:d]
    w1b = W1[d:]

    idx_t, q, p = _run_knn_qp(pos, batch_i32, x, w1a, w1b, b1,
                              n=n, h=h, r=min(256, n), c=min(256, n))

    if n % 2 == 0 and (KP * (n // 2)) % (SC_WORKERS * 512) == 0:
        n2 = n // 2
        halves = []
        for lo_i in (0, n2):
            src_h = idx_t[:, lo_i:lo_i + n2].reshape(-1)
            qe_h = _sc_gather(q, src_h, n=n, h=h)
            halves.append(
                _run_mlp_max(qe_h.reshape(KP, n2, h), p[lo_i:lo_i + n2],
                             W2, b2, n=n2, h=h, t=min(128, n2)))
        out = jnp.concatenate(halves, axis=0)
    else:
        src_flat = idx_t.reshape(-1)
        qe = _sc_gather(q, src_flat, n=n, h=h)
        out = _run_mlp_max(qe.reshape(KP, n, h), p, W2, b2, n=n, h=h,
                           t=min(128, n))

    src = idx_t[:K].T.reshape(-1)
    dst = jnp.broadcast_to(jnp.arange(n, dtype=jnp.int32)[:, None],
                           (n, K)).reshape(-1)
    edge_index = jnp.stack([src, dst])
    return (out, pos, batch, edge_index)
